# initial kernel scaffold (unmeasured)
import jax
import jax.numpy as jnp
from jax import lax
from jax.experimental import pallas as pl
from jax.experimental.pallas import tpu as pltpu

P = 8
M_PER = 512
N_PER = 1024
K = 4096
CH = 512
NCH = N_PER // CH


def kernel(x, w_mat):
    def body(x_ref, w_ref, out_ref, xbf, wbuf, sendbuf, recvbuf,
             wdma_sems, send_sems, recv_sems):
        my = lax.axis_index("i")

        bsem = pltpu.get_barrier_semaphore()
        for p in range(P):
            @pl.when(my != p)
            def _():
                pl.semaphore_signal(
                    bsem, inc=1, device_id=(p,),
                    device_id_type=pl.DeviceIdType.MESH,
                )
        pl.semaphore_wait(bsem, P - 1)

        xbf[...] = x_ref[...].astype(jnp.bfloat16)

        n_chunks = P * NCH

        def w_dma(g, slot):
            t = (my + g // NCH) % P
            col = t * N_PER + (g % NCH) * CH
            return pltpu.make_async_copy(
                w_ref.at[:, pl.ds(col, CH)], wbuf.at[slot], wdma_sems.at[slot]
            )

        w_dma(0, 0).start()

        sends = {}
        for g in range(n_chunks):
            s, c = g // NCH, g % NCH
            if g + 1 < n_chunks:
                w_dma(g + 1, (g + 1) % 2).start()
            w_dma(g, g % 2).wait()

            y = jnp.dot(
                xbf[...], wbuf[g % 2].astype(jnp.bfloat16),
                preferred_element_type=jnp.float32,
            )
            y = jnp.maximum(y, 0.0)

            if s == 0:
                out_ref[pl.ds(my * M_PER, M_PER), c * CH:(c + 1) * CH] = y
            else:
                if c == 0 and s >= 3:
                    sends[s - 2].wait_send()
                sendbuf[s % 2, :, c * CH:(c + 1) * CH] = y.astype(jnp.bfloat16)
                if c == NCH - 1:
                    t = (my + s) % P
                    rdma = pltpu.make_async_remote_copy(
                        src_ref=sendbuf.at[s % 2],
                        dst_ref=recvbuf.at[s],
                        send_sem=send_sems.at[s % 2],
                        recv_sem=recv_sems.at[s],
                        device_id=(t,),
                        device_id_type=pl.DeviceIdType.MESH,
                    )
                    rdma.start()
                    sends[s] = rdma

        sends[P - 2].wait_send()
        sends[P - 1].wait_send()

        for s in range(1, P):
            recv = pltpu.make_async_remote_copy(
                src_ref=sendbuf.at[0],
                dst_ref=recvbuf.at[s],
                send_sem=send_sems.at[0],
                recv_sem=recv_sems.at[s],
                device_id=(0,),
                device_id_type=pl.DeviceIdType.MESH,
            )
            recv.wait_recv()
            src = (my - s) % P
            out_ref[pl.ds(src * M_PER, M_PER), :] = (
                recvbuf[s].astype(jnp.float32)
            )

    return pl.pallas_call(
        body,
        out_shape=jax.ShapeDtypeStruct((P * M_PER, N_PER), jnp.float32),
        in_specs=[
            pl.BlockSpec(memory_space=pltpu.VMEM),
            pl.BlockSpec(memory_space=pltpu.ANY),
        ],
        out_specs=pl.BlockSpec(memory_space=pltpu.VMEM),
        scratch_shapes=[
            pltpu.VMEM((M_PER, K), jnp.bfloat16),
            pltpu.VMEM((2, K, CH), jnp.float32),
            pltpu.VMEM((2, M_PER, N_PER), jnp.bfloat16),
            pltpu.VMEM((P, M_PER, N_PER), jnp.bfloat16),
            pltpu.SemaphoreType.DMA((2,)),
            pltpu.SemaphoreType.DMA((2,)),
            pltpu.SemaphoreType.DMA((P,)),
        ],
        compiler_params=pltpu.CompilerParams(collective_id=0),
    )(x, w_mat)


# baseline (device time: 143250 ns/iter reference)
import jax
import jax.numpy as jnp
from jax import lax
from jax.experimental import pallas as pl
from jax.experimental.pallas import tpu as pltpu

P = 8
M_PER = 512
N_PER = 1024
K = 4096
CH = 256
NCH = N_PER // CH


def kernel(x, w_mat):
    def body(x_ref, w_ref, out_ref, xbf, wbuf, sendbuf, recvbuf,
             wdma_sems, send_sems, recv_sems):
        my = lax.axis_index("i")

        bsem = pltpu.get_barrier_semaphore()
        for p in range(P):
            @pl.when(my != p)
            def _():
                pl.semaphore_signal(
                    bsem, inc=1, device_id=(p,),
                    device_id_type=pl.DeviceIdType.MESH,
                )
        pl.semaphore_wait(bsem, P - 1)

        xbf[...] = x_ref[...].astype(jnp.bfloat16)

        n_chunks = P * NCH

        def w_dma(g, slot):
            t = (my + g // NCH) % P
            col = t * N_PER + (g % NCH) * CH
            return pltpu.make_async_copy(
                w_ref.at[:, pl.ds(col, CH)], wbuf.at[slot], wdma_sems.at[slot]
            )

        w_dma(0, 0).start()

        sends = {}
        for g in range(n_chunks):
            s, c = g // NCH, g % NCH
            if g + 1 < n_chunks:
                w_dma(g + 1, (g + 1) % 2).start()
            w_dma(g, g % 2).wait()

            y = jnp.dot(
                xbf[...], wbuf[g % 2].astype(jnp.bfloat16),
                preferred_element_type=jnp.float32,
            )
            y = jnp.maximum(y, 0.0)

            if s == 0:
                out_ref[pl.ds(my * M_PER, M_PER), c * CH:(c + 1) * CH] = y
            else:
                if c == 0 and s >= 3:
                    sends[s - 2].wait_send()
                sendbuf[s % 2, :, c * CH:(c + 1) * CH] = y.astype(jnp.bfloat16)
                if c == NCH - 1:
                    t = (my + s) % P
                    rdma = pltpu.make_async_remote_copy(
                        src_ref=sendbuf.at[s % 2],
                        dst_ref=recvbuf.at[s - 1],
                        send_sem=send_sems.at[s % 2],
                        recv_sem=recv_sems.at[s],
                        device_id=(t,),
                        device_id_type=pl.DeviceIdType.MESH,
                    )
                    rdma.start()
                    sends[s] = rdma

        sends[P - 2].wait_send()
        sends[P - 1].wait_send()

        for s in range(1, P):
            recv = pltpu.make_async_remote_copy(
                src_ref=sendbuf.at[0],
                dst_ref=recvbuf.at[s - 1],
                send_sem=send_sems.at[0],
                recv_sem=recv_sems.at[s],
                device_id=(0,),
                device_id_type=pl.DeviceIdType.MESH,
            )
            recv.wait_recv()
            src = (my - s) % P
            out_ref[pl.ds(src * M_PER, M_PER), :] = (
                recvbuf[s - 1].astype(jnp.float32)
            )

    return pl.pallas_call(
        body,
        out_shape=jax.ShapeDtypeStruct((P * M_PER, N_PER), jnp.float32),
        in_specs=[
            pl.BlockSpec(memory_space=pltpu.VMEM),
            pl.BlockSpec(memory_space=pl.ANY),
        ],
        out_specs=pl.BlockSpec(memory_space=pltpu.VMEM),
        scratch_shapes=[
            pltpu.VMEM((M_PER, K), jnp.bfloat16),
            pltpu.VMEM((2, K, CH), jnp.float32),
            pltpu.VMEM((2, M_PER, N_PER), jnp.bfloat16),
            pltpu.VMEM((P - 1, M_PER, N_PER), jnp.bfloat16),
            pltpu.SemaphoreType.DMA((2,)),
            pltpu.SemaphoreType.DMA((2,)),
            pltpu.SemaphoreType.DMA((P,)),
        ],
        compiler_params=pltpu.CompilerParams(
            collective_id=0,
            vmem_limit_bytes=100 * 1024 * 1024,
        ),
    )(x, w_mat)


# device time: 124417 ns/iter; 1.1514x vs baseline; 1.1514x over previous
import jax
import jax.numpy as jnp
from jax import lax
from jax.experimental import pallas as pl
from jax.experimental.pallas import tpu as pltpu

P = 8
M_PER = 512
N_PER = 1024
K = 4096
CH = 512
NCH = N_PER // CH
NSUB = 4
SUBK = K // NSUB


def kernel(x, w_mat):
    def body(x_ref, w_ref, out_ref, wbuf, sendbuf, recvbuf,
             wdma_sems, send_sems, recv_sems):
        my = lax.axis_index("i")
        n_chunks = P * NCH

        def w_dma(g, slot, sub):
            t = (my + g // NCH) % P
            col = t * N_PER + (g % NCH) * CH
            return pltpu.make_async_copy(
                w_ref.at[sub * SUBK:(sub + 1) * SUBK, pl.ds(col, CH)],
                wbuf.at[slot, sub * SUBK:(sub + 1) * SUBK, :],
                wdma_sems.at[slot, sub],
            )

        for sub in range(NSUB):
            w_dma(0, 0, sub).start()

        bsem = pltpu.get_barrier_semaphore()
        for p in range(P):
            @pl.when(my != p)
            def _():
                pl.semaphore_signal(
                    bsem, inc=1, device_id=(p,),
                    device_id_type=pl.DeviceIdType.MESH,
                )
        pl.semaphore_wait(bsem, P - 1)

        def recv_slot(s):
            recv = pltpu.make_async_remote_copy(
                src_ref=sendbuf.at[0],
                dst_ref=recvbuf.at[s - 1],
                send_sem=send_sems.at[0],
                recv_sem=recv_sems.at[s],
                device_id=(0,),
                device_id_type=pl.DeviceIdType.MESH,
            )
            recv.wait_recv()
            src = (my - s) % P
            out_ref[pl.ds(src * M_PER, M_PER), :] = (
                recvbuf[s - 1].astype(jnp.float32)
            )

        sends = {}
        done_recv = 0
        for g in range(n_chunks):
            s, c = g // NCH, g % NCH
            if g + 1 < n_chunks:
                for sub in range(NSUB):
                    w_dma(g + 1, (g + 1) % 2, sub).start()

            if c == 0 and 3 <= s and done_recv < s - 2:
                done_recv += 1
                recv_slot(done_recv)

            for sub in range(NSUB):
                w_dma(g, g % 2, sub).wait()

            y = lax.dot_general(
                x_ref[...], wbuf[g % 2],
                (((1,), (0,)), ((), ())),
                precision=lax.Precision.DEFAULT,
                preferred_element_type=jnp.float32,
            )
            y = jnp.maximum(y, 0.0)

            if s == 0:
                out_ref[pl.ds(my * M_PER, M_PER), c * CH:(c + 1) * CH] = y
            else:
                if c == 0 and s >= 3:
                    sends[s - 2].wait_send()
                sendbuf[s % 2, :, c * CH:(c + 1) * CH] = y.astype(jnp.bfloat16)
                if c == NCH - 1:
                    t = (my + s) % P
                    rdma = pltpu.make_async_remote_copy(
                        src_ref=sendbuf.at[s % 2],
                        dst_ref=recvbuf.at[s - 1],
                        send_sem=send_sems.at[s % 2],
                        recv_sem=recv_sems.at[s],
                        device_id=(t,),
                        device_id_type=pl.DeviceIdType.MESH,
                    )
                    rdma.start()
                    sends[s] = rdma

        sends[P - 2].wait_send()
        sends[P - 1].wait_send()

        for s in range(done_recv + 1, P):
            recv_slot(s)

    return pl.pallas_call(
        body,
        out_shape=jax.ShapeDtypeStruct((P * M_PER, N_PER), jnp.float32),
        in_specs=[
            pl.BlockSpec(memory_space=pltpu.VMEM),
            pl.BlockSpec(memory_space=pl.ANY),
        ],
        out_specs=pl.BlockSpec(memory_space=pltpu.VMEM),
        scratch_shapes=[
            pltpu.VMEM((2, K, CH), jnp.float32),
            pltpu.VMEM((2, M_PER, N_PER), jnp.bfloat16),
            pltpu.VMEM((P - 1, M_PER, N_PER), jnp.bfloat16),
            pltpu.SemaphoreType.DMA((2, NSUB)),
            pltpu.SemaphoreType.DMA((2,)),
            pltpu.SemaphoreType.DMA((P,)),
        ],
        compiler_params=pltpu.CompilerParams(
            collective_id=0,
            vmem_limit_bytes=100 * 1024 * 1024,
        ),
    )(x, w_mat)


# device time: 100115 ns/iter; 1.4309x vs baseline; 1.2427x over previous
import jax
import jax.numpy as jnp
from jax import lax
from jax.experimental import pallas as pl
from jax.experimental.pallas import tpu as pltpu

P = 8
M_PER = 512
N_PER = 1024
K = 4096
CH = 512
NCH = N_PER // CH
NSUB = 4
SUBK = K // NSUB


def kernel(x, w_mat):
    def body(x_ref, w_ref, out_ref, wbuf, sendbuf, recvbuf,
             wdma_sems, send_sems, recv_sems):
        my = lax.axis_index("i")
        n_chunks = P * NCH

        def w_dma(g, slot, sub):
            t = (my + g // NCH) % P
            col = t * N_PER + (g % NCH) * CH
            return pltpu.make_async_copy(
                w_ref.at[sub * SUBK:(sub + 1) * SUBK, pl.ds(col, CH)],
                wbuf.at[slot, sub * SUBK:(sub + 1) * SUBK, :],
                wdma_sems.at[slot, sub],
            )

        for sub in range(NSUB):
            w_dma(0, 0, sub).start()

        bsem = pltpu.get_barrier_semaphore()
        for p in range(P):
            @pl.when(my != p)
            def _():
                pl.semaphore_signal(
                    bsem, inc=1, device_id=(p,),
                    device_id_type=pl.DeviceIdType.MESH,
                )
        pl.semaphore_wait(bsem, P - 1)

        def half_rdma(s, c, device_id):
            return pltpu.make_async_remote_copy(
                src_ref=sendbuf.at[s - 1, :, c * CH:(c + 1) * CH],
                dst_ref=recvbuf.at[s - 1, :, c * CH:(c + 1) * CH],
                send_sem=send_sems.at[s - 1, c],
                recv_sem=recv_sems.at[s - 1, c],
                device_id=device_id,
                device_id_type=pl.DeviceIdType.MESH,
            )

        sends = []
        for g in range(n_chunks):
            s, c = g // NCH, g % NCH
            if g + 1 < n_chunks:
                for sub in range(NSUB):
                    w_dma(g + 1, (g + 1) % 2, sub).start()
            for sub in range(NSUB):
                w_dma(g, g % 2, sub).wait()

            y = lax.dot_general(
                x_ref[...], wbuf[g % 2],
                (((1,), (0,)), ((), ())),
                precision=lax.Precision.DEFAULT,
                preferred_element_type=jnp.float32,
            )
            y = jnp.maximum(y, 0.0)

            if s == 0:
                out_ref[pl.ds(my * M_PER, M_PER), c * CH:(c + 1) * CH] = y
            else:
                sendbuf[s - 1, :, c * CH:(c + 1) * CH] = y.astype(jnp.bfloat16)
                rdma = half_rdma(s, c, ((my + s) % P,))
                rdma.start()
                sends.append(rdma)

        for s in range(1, P):
            for c in range(NCH):
                half_rdma(s, c, (0,)).wait_recv()
            src = (my - s) % P
            out_ref[pl.ds(src * M_PER, M_PER), :] = (
                recvbuf[s - 1].astype(jnp.float32)
            )

        for rdma in sends:
            rdma.wait_send()

    return pl.pallas_call(
        body,
        out_shape=jax.ShapeDtypeStruct((P * M_PER, N_PER), jnp.float32),
        in_specs=[
            pl.BlockSpec(memory_space=pltpu.VMEM),
            pl.BlockSpec(memory_space=pl.ANY),
        ],
        out_specs=pl.BlockSpec(memory_space=pltpu.VMEM),
        scratch_shapes=[
            pltpu.VMEM((2, K, CH), jnp.float32),
            pltpu.VMEM((P - 1, M_PER, N_PER), jnp.bfloat16),
            pltpu.VMEM((P - 1, M_PER, N_PER), jnp.bfloat16),
            pltpu.SemaphoreType.DMA((2, NSUB)),
            pltpu.SemaphoreType.DMA((P - 1, NCH)),
            pltpu.SemaphoreType.DMA((P - 1, NCH)),
        ],
        compiler_params=pltpu.CompilerParams(
            collective_id=0,
            vmem_limit_bytes=100 * 1024 * 1024,
        ),
    )(x, w_mat)
